# initial kernel scaffold (unmeasured)
import jax
import jax.numpy as jnp
from jax import lax
from jax.experimental import pallas as pl
from jax.experimental.pallas import tpu as pltpu

N_DEV = 4
N_TOK = 2048
D_MODEL = 512
D_FF = 1024
E_LOCAL = 8
CHUNK = N_TOK // N_DEV


def kernel(x, router_W, route_idx, expert_W, shared_W):
    def body(
        x_ref,
        router_ref,
        route_ref,
        ew_ref,
        sw_ref,
        out_ref,
        rs_buf,
        rs_send_sems,
        rs_recv_sems,
        ag_send_sems,
        ag_recv_sems,
    ):
        my_pos = lax.axis_index("i")
        left = lax.rem(my_pos - 1 + N_DEV, N_DEV)
        right = lax.rem(my_pos + 1, N_DEV)

        barrier_sem = pltpu.get_barrier_semaphore()
        for nbr in (left, right):
            pl.semaphore_signal(
                barrier_sem,
                inc=1,
                device_id=(nbr,),
                device_id_type=pl.DeviceIdType.MESH,
            )
        pl.semaphore_wait(barrier_sem, 2)

        xv = x_ref[:, :]
        scores = jnp.dot(
            xv, router_ref[:, :], preferred_element_type=jnp.float32
        )
        s_max = jnp.max(scores, axis=-1, keepdims=True)
        p_sel = 1.0 / jnp.sum(jnp.exp(scores - s_max), axis=-1, keepdims=True)
        route = route_ref[:, :]

        acc = jnp.zeros((N_TOK, D_FF), jnp.float32)
        for j in range(E_LOCAL):
            e = my_pos * E_LOCAL + j
            w = jnp.where(route == e, p_sel, 0.0)
            acc = acc + jnp.dot(
                xv * w, ew_ref[j], preferred_element_type=jnp.float32
            )
        out_ref[:, :] = acc

        for h in range(N_DEV - 1):
            send_chunk = lax.rem(my_pos - h + N_DEV, N_DEV)
            rdma = pltpu.make_async_remote_copy(
                src_ref=out_ref.at[pl.ds(send_chunk * CHUNK, CHUNK), :],
                dst_ref=rs_buf.at[h],
                send_sem=rs_send_sems.at[h],
                recv_sem=rs_recv_sems.at[h],
                device_id=(right,),
                device_id_type=pl.DeviceIdType.MESH,
            )
            rdma.start()
            rdma.wait()
            add_chunk = lax.rem(my_pos - h - 1 + N_DEV, N_DEV)
            sl = pl.ds(add_chunk * CHUNK, CHUNK)
            out_ref[sl, :] = out_ref[sl, :] + rs_buf[h]

        for g in range(N_DEV - 1):
            send_chunk = lax.rem(my_pos + 1 - g + N_DEV, N_DEV)
            sl = pl.ds(send_chunk * CHUNK, CHUNK)
            rdma = pltpu.make_async_remote_copy(
                src_ref=out_ref.at[sl, :],
                dst_ref=out_ref.at[sl, :],
                send_sem=ag_send_sems.at[g],
                recv_sem=ag_recv_sems.at[g],
                device_id=(right,),
                device_id_type=pl.DeviceIdType.MESH,
            )
            rdma.start()
            rdma.wait()

        out_ref[:, :] = out_ref[:, :] + jnp.dot(
            xv, sw_ref[:, :], preferred_element_type=jnp.float32
        )

    return pl.pallas_call(
        body,
        out_shape=jax.ShapeDtypeStruct((N_TOK, D_FF), jnp.float32),
        in_specs=[
            pl.BlockSpec(memory_space=pltpu.VMEM),
            pl.BlockSpec(memory_space=pltpu.VMEM),
            pl.BlockSpec(memory_space=pltpu.VMEM),
            pl.BlockSpec(memory_space=pltpu.VMEM),
            pl.BlockSpec(memory_space=pltpu.VMEM),
        ],
        out_specs=pl.BlockSpec(memory_space=pltpu.VMEM),
        scratch_shapes=[
            pltpu.VMEM((N_DEV - 1, CHUNK, D_FF), jnp.float32),
            pltpu.SemaphoreType.DMA((N_DEV - 1,)),
            pltpu.SemaphoreType.DMA((N_DEV - 1,)),
            pltpu.SemaphoreType.DMA((N_DEV - 1,)),
            pltpu.SemaphoreType.DMA((N_DEV - 1,)),
        ],
        compiler_params=pltpu.CompilerParams(collective_id=0),
    )(x, router_W, route_idx, expert_W, shared_W)


# baseline (device time: 191971 ns/iter reference)
import jax
import jax.numpy as jnp
from jax import lax
from jax.experimental import pallas as pl
from jax.experimental.pallas import tpu as pltpu

N_DEV = 4
N_TOK = 2048
D_MODEL = 512
D_FF = 1024
E_LOCAL = 8
CHUNK = N_TOK // N_DEV


def kernel(x, router_W, route_idx, expert_W, shared_W):
    def body(
        x_ref,
        router_ref,
        route_ref,
        ew_ref,
        sw_ref,
        out_ref,
        rs_buf,
        rs_send_sems,
        rs_recv_sems,
        ag_send_sems,
        ag_recv_sems,
    ):
        my_pos = lax.axis_index("i")
        left = lax.rem(my_pos - 1 + N_DEV, N_DEV)
        right = lax.rem(my_pos + 1, N_DEV)

        barrier_sem = pltpu.get_barrier_semaphore()
        for nbr in (left, right):
            pl.semaphore_signal(
                barrier_sem,
                inc=1,
                device_id=(nbr,),
                device_id_type=pl.DeviceIdType.MESH,
            )
        pl.semaphore_wait(barrier_sem, 2)

        xv = x_ref[:, :]
        scores = jnp.dot(
            xv, router_ref[:, :], preferred_element_type=jnp.float32
        )
        s_max = jnp.max(scores, axis=-1, keepdims=True)
        p_sel = 1.0 / jnp.sum(jnp.exp(scores - s_max), axis=-1, keepdims=True)
        route = route_ref[:, :]

        for j in range(E_LOCAL):
            e = my_pos * E_LOCAL + j
            w = jnp.where(route == e, p_sel, 0.0)
            part = jnp.dot(
                xv * w, ew_ref[j], preferred_element_type=jnp.float32
            )
            if j == 0:
                out_ref[:, :] = part
            else:
                out_ref[:, :] = out_ref[:, :] + part

        for h in range(N_DEV - 1):
            send_chunk = lax.rem(my_pos - h + N_DEV, N_DEV)
            rdma = pltpu.make_async_remote_copy(
                src_ref=out_ref.at[pl.ds(send_chunk * CHUNK, CHUNK), :],
                dst_ref=rs_buf.at[h],
                send_sem=rs_send_sems.at[h],
                recv_sem=rs_recv_sems.at[h],
                device_id=(right,),
                device_id_type=pl.DeviceIdType.MESH,
            )
            rdma.start()
            rdma.wait()
            add_chunk = lax.rem(my_pos - h - 1 + N_DEV, N_DEV)
            sl = pl.ds(add_chunk * CHUNK, CHUNK)
            out_ref[sl, :] = out_ref[sl, :] + rs_buf[h]

        for g in range(N_DEV - 1):
            send_chunk = lax.rem(my_pos + 1 - g + N_DEV, N_DEV)
            sl = pl.ds(send_chunk * CHUNK, CHUNK)
            rdma = pltpu.make_async_remote_copy(
                src_ref=out_ref.at[sl, :],
                dst_ref=out_ref.at[sl, :],
                send_sem=ag_send_sems.at[g],
                recv_sem=ag_recv_sems.at[g],
                device_id=(right,),
                device_id_type=pl.DeviceIdType.MESH,
            )
            rdma.start()
            rdma.wait()

        out_ref[:, :] = out_ref[:, :] + jnp.dot(
            xv, sw_ref[:, :], preferred_element_type=jnp.float32
        )

    return pl.pallas_call(
        body,
        out_shape=jax.ShapeDtypeStruct((N_TOK, D_FF), jnp.float32),
        in_specs=[
            pl.BlockSpec(memory_space=pltpu.VMEM),
            pl.BlockSpec(memory_space=pltpu.VMEM),
            pl.BlockSpec(memory_space=pltpu.VMEM),
            pl.BlockSpec(memory_space=pltpu.VMEM),
            pl.BlockSpec(memory_space=pltpu.VMEM),
        ],
        out_specs=pl.BlockSpec(memory_space=pltpu.VMEM),
        scratch_shapes=[
            pltpu.VMEM((N_DEV - 1, CHUNK, D_FF), jnp.float32),
            pltpu.SemaphoreType.DMA((N_DEV - 1,)),
            pltpu.SemaphoreType.DMA((N_DEV - 1,)),
            pltpu.SemaphoreType.DMA((N_DEV - 1,)),
            pltpu.SemaphoreType.DMA((N_DEV - 1,)),
        ],
        compiler_params=pltpu.CompilerParams(
            collective_id=0,
            vmem_limit_bytes=100 * 1024 * 1024,
        ),
    )(x, router_W, route_idx, expert_W, shared_W)


# device time: 114571 ns/iter; 1.6756x vs baseline; 1.6756x over previous
import jax
import jax.numpy as jnp
from jax import lax
from jax.experimental import pallas as pl
from jax.experimental.pallas import tpu as pltpu

N_DEV = 4
N_TOK = 2048
D_MODEL = 512
D_FF = 1024
E_LOCAL = 8
CHUNK = N_TOK // N_DEV
BF = jnp.bfloat16


def kernel(x, router_W, route_idx, expert_W, shared_W):
    def body(
        x_ref,
        router_ref,
        route_ref,
        ew_ref,
        sw_ref,
        out_ref,
        p_ref,
        x_bf,
        ew_bf,
        sw_bf,
        snd_buf,
        rs_buf,
        red_buf,
        ag_buf,
        rs_send_sems,
        rs_recv_sems,
        ag_send_sems,
        ag_recv_sems,
    ):
        my_pos = lax.axis_index("i")
        left = lax.rem(my_pos - 1 + N_DEV, N_DEV)
        right = lax.rem(my_pos + 1, N_DEV)

        barrier_sem = pltpu.get_barrier_semaphore()
        for nbr in (left, right):
            pl.semaphore_signal(
                barrier_sem,
                inc=1,
                device_id=(nbr,),
                device_id_type=pl.DeviceIdType.MESH,
            )
        pl.semaphore_wait(barrier_sem, 2)

        xv = x_ref[:, :]
        scores = jnp.dot(
            xv, router_ref[:, :], preferred_element_type=jnp.float32
        )
        s_max = jnp.max(scores, axis=-1, keepdims=True)
        p_ref[:, :] = 1.0 / jnp.sum(
            jnp.exp(scores - s_max), axis=-1, keepdims=True
        )

        x_bf[:, :] = xv.astype(BF)
        sw_bf[:, :] = sw_ref[:, :].astype(BF)
        for j in range(E_LOCAL):
            ew_bf[j, :, :] = ew_ref[j].astype(BF)

        def chunk_partial(c):
            sl = pl.ds(c * CHUNK, CHUNK)
            xc = x_bf[sl, :]
            routec = route_ref[sl, :]
            pc = p_ref[sl, :]
            acc = jnp.zeros((CHUNK, D_FF), jnp.float32)
            for j in range(E_LOCAL):
                e = my_pos * E_LOCAL + j
                wj = jnp.where(routec == e, pc, 0.0).astype(BF)
                acc = acc + jnp.dot(
                    xc * wj, ew_bf[j], preferred_element_type=jnp.float32
                )
            return acc

        def shared_chunk(c):
            sl = pl.ds(c * CHUNK, CHUNK)
            return jnp.dot(
                x_bf[sl, :], sw_bf[:, :], preferred_element_type=jnp.float32
            )

        def hop(src_ref, dst_ref, send_sem, recv_sem):
            return pltpu.make_async_remote_copy(
                src_ref=src_ref,
                dst_ref=dst_ref,
                send_sem=send_sem,
                recv_sem=recv_sem,
                device_id=(right,),
                device_id_type=pl.DeviceIdType.MESH,
            )

        c_own = right

        snd_buf[0, :, :] = chunk_partial(my_pos).astype(BF)
        rs = []
        rs.append(hop(snd_buf.at[0], rs_buf.at[0],
                      rs_send_sems.at[0], rs_recv_sems.at[0]))
        rs[0].start()

        part1 = chunk_partial(lax.rem(my_pos - 1 + N_DEV, N_DEV))
        rs[0].wait_recv()
        snd_buf[1, :, :] = (part1 + rs_buf[0][:, :].astype(jnp.float32)).astype(BF)
        rs.append(hop(snd_buf.at[1], rs_buf.at[1],
                      rs_send_sems.at[1], rs_recv_sems.at[1]))
        rs[1].start()

        part2 = chunk_partial(lax.rem(my_pos - 2 + N_DEV, N_DEV))
        rs[1].wait_recv()
        snd_buf[2, :, :] = (part2 + rs_buf[1][:, :].astype(jnp.float32)).astype(BF)
        rs.append(hop(snd_buf.at[2], rs_buf.at[2],
                      rs_send_sems.at[2], rs_recv_sems.at[2]))
        rs[2].start()

        part3 = chunk_partial(c_own)
        sh_own = shared_chunk(c_own)
        rs[2].wait_recv()
        red = part3 + rs_buf[2][:, :].astype(jnp.float32)
        red_buf[:, :] = red.astype(BF)

        ag = []
        ag.append(hop(red_buf, ag_buf.at[0],
                      ag_send_sems.at[0], ag_recv_sems.at[0]))
        ag[0].start()
        out_ref[pl.ds(c_own * CHUNK, CHUNK), :] = red + sh_own

        ag[0].wait_recv()
        ag.append(hop(ag_buf.at[0], ag_buf.at[1],
                      ag_send_sems.at[1], ag_recv_sems.at[1]))
        ag[1].start()
        out_ref[pl.ds(my_pos * CHUNK, CHUNK), :] = (
            ag_buf[0][:, :].astype(jnp.float32) + shared_chunk(my_pos)
        )

        ag[1].wait_recv()
        ag.append(hop(ag_buf.at[1], ag_buf.at[2],
                      ag_send_sems.at[2], ag_recv_sems.at[2]))
        ag[2].start()
        c = lax.rem(my_pos - 1 + N_DEV, N_DEV)
        out_ref[pl.ds(c * CHUNK, CHUNK), :] = (
            ag_buf[1][:, :].astype(jnp.float32) + shared_chunk(c)
        )

        ag[2].wait_recv()
        c = lax.rem(my_pos - 2 + N_DEV, N_DEV)
        out_ref[pl.ds(c * CHUNK, CHUNK), :] = (
            ag_buf[2][:, :].astype(jnp.float32) + shared_chunk(c)
        )

        for r in rs:
            r.wait_send()
        for a in ag:
            a.wait_send()

    return pl.pallas_call(
        body,
        out_shape=jax.ShapeDtypeStruct((N_TOK, D_FF), jnp.float32),
        in_specs=[
            pl.BlockSpec(memory_space=pltpu.VMEM),
            pl.BlockSpec(memory_space=pltpu.VMEM),
            pl.BlockSpec(memory_space=pltpu.VMEM),
            pl.BlockSpec(memory_space=pltpu.VMEM),
            pl.BlockSpec(memory_space=pltpu.VMEM),
        ],
        out_specs=pl.BlockSpec(memory_space=pltpu.VMEM),
        scratch_shapes=[
            pltpu.VMEM((N_TOK, 1), jnp.float32),
            pltpu.VMEM((N_TOK, D_MODEL), BF),
            pltpu.VMEM((E_LOCAL, D_MODEL, D_FF), BF),
            pltpu.VMEM((D_MODEL, D_FF), BF),
            pltpu.VMEM((N_DEV - 1, CHUNK, D_FF), BF),
            pltpu.VMEM((N_DEV - 1, CHUNK, D_FF), BF),
            pltpu.VMEM((CHUNK, D_FF), BF),
            pltpu.VMEM((N_DEV - 1, CHUNK, D_FF), BF),
            pltpu.SemaphoreType.DMA((N_DEV - 1,)),
            pltpu.SemaphoreType.DMA((N_DEV - 1,)),
            pltpu.SemaphoreType.DMA((N_DEV - 1,)),
            pltpu.SemaphoreType.DMA((N_DEV - 1,)),
        ],
        compiler_params=pltpu.CompilerParams(
            collective_id=0,
            vmem_limit_bytes=100 * 1024 * 1024,
        ),
    )(x, router_W, route_idx, expert_W, shared_W)


# device time: 85326 ns/iter; 2.2499x vs baseline; 1.3427x over previous
import jax
import jax.numpy as jnp
from jax import lax
from jax.experimental import pallas as pl
from jax.experimental.pallas import tpu as pltpu

N_DEV = 4
N_TOK = 2048
D_MODEL = 512
D_FF = 1024
E_LOCAL = 8
CHUNK = N_TOK // N_DEV
HF = D_FF // 2
BF = jnp.bfloat16
F32 = jnp.float32


def kernel(x, router_W, route_idx, expert_W, shared_W):
    def body(
        x_ref,
        router_ref,
        route_ref,
        ew_ref,
        sw_ref,
        out_ref,
        p_ref,
        x_bf,
        sw_bf,
        sndR, rsR, redR_buf, agR,
        sndL, rsL, redL_buf, agL,
        sems,
    ):
        my_pos = lax.axis_index("i")
        left = lax.rem(my_pos - 1 + N_DEV, N_DEV)
        right = lax.rem(my_pos + 1, N_DEV)

        def cmod(k):
            return lax.rem(my_pos + k + N_DEV, N_DEV)

        barrier_sem = pltpu.get_barrier_semaphore()
        for nbr in (left, right):
            pl.semaphore_signal(
                barrier_sem,
                inc=1,
                device_id=(nbr,),
                device_id_type=pl.DeviceIdType.MESH,
            )
        pl.semaphore_wait(barrier_sem, 2)

        xv = x_ref[:, :]
        scores = jnp.dot(xv, router_ref[:, :], preferred_element_type=F32)
        s_max = jnp.max(scores, axis=-1, keepdims=True)
        p_ref[:, :] = 1.0 / jnp.sum(
            jnp.exp(scores - s_max), axis=-1, keepdims=True
        )

        x_bf[:, :] = xv.astype(BF)
        sw_bf[:, :] = sw_ref[:, :].astype(BF)

        def chunk_partial(c):
            sl = pl.ds(c * CHUNK, CHUNK)
            xc = x_bf[sl, :]
            routec = route_ref[sl, :]
            pc = p_ref[sl, :]
            acc = jnp.zeros((CHUNK, D_FF), F32)
            for j in range(E_LOCAL):
                e = my_pos * E_LOCAL + j
                wj = jnp.where(routec == e, pc, 0.0).astype(BF)
                acc = acc + jnp.dot(
                    xc * wj, ew_ref[j].astype(BF), preferred_element_type=F32
                )
            return acc[:, :HF].astype(BF), acc[:, HF:].astype(BF)

        def shared_half(c, lo):
            sl = pl.ds(c * CHUNK, CHUNK)
            return jnp.dot(
                x_bf[sl, :], sw_bf[:, lo:lo + HF],
                preferred_element_type=F32,
            )

        def hop(src_ref, dst_ref, send_sem, recv_sem, to):
            return pltpu.make_async_remote_copy(
                src_ref=src_ref,
                dst_ref=dst_ref,
                send_sem=send_sem,
                recv_sem=recv_sem,
                device_id=(to,),
                device_id_type=pl.DeviceIdType.MESH,
            )

        def hopR(h):
            return hop(sndR.at[h], rsR.at[h], sems.at[0, h], sems.at[1, h],
                       right)

        def hopL(h):
            return hop(sndL.at[h], rsL.at[h], sems.at[4, h], sems.at[5, h],
                       left)

        L, R = 0, HF

        ppL, ppR = chunk_partial(my_pos)
        sndR[0, :, :] = ppR
        sndL[0, :, :] = ppL
        r0 = hopR(0)
        l0 = hopL(0)
        r0.start()
        l0.start()

        pm1L, pm1R = chunk_partial(cmod(-1))
        pp1L, pp1R = chunk_partial(cmod(+1))

        r0.wait_recv()
        sndR[1, :, :] = (
            pm1R.astype(F32) + rsR[0][:, :].astype(F32)
        ).astype(BF)
        r1 = hopR(1)
        r1.start()
        l0.wait_recv()
        sndL[1, :, :] = (
            pp1L.astype(F32) + rsL[0][:, :].astype(F32)
        ).astype(BF)
        l1 = hopL(1)
        l1.start()

        pp2L, pp2R = chunk_partial(cmod(+2))

        r1.wait_recv()
        sndR[2, :, :] = (
            pp2R.astype(F32) + rsR[1][:, :].astype(F32)
        ).astype(BF)
        r2 = hopR(2)
        r2.start()
        l1.wait_recv()
        sndL[2, :, :] = (
            pp2L.astype(F32) + rsL[1][:, :].astype(F32)
        ).astype(BF)
        l2 = hopL(2)
        l2.start()

        shR_own = shared_half(cmod(+1), R)
        shL_own = shared_half(cmod(-1), L)

        r2.wait_recv()
        redR = pp1R.astype(F32) + rsR[2][:, :].astype(F32)
        redR_buf[:, :] = redR.astype(BF)
        l2.wait_recv()
        redL = pm1L.astype(F32) + rsL[2][:, :].astype(F32)
        redL_buf[:, :] = redL.astype(BF)

        ar0 = hop(redR_buf, agR.at[0], sems.at[2, 0], sems.at[3, 0], right)
        al0 = hop(redL_buf, agL.at[0], sems.at[6, 0], sems.at[7, 0], left)
        ar0.start()
        al0.start()
        out_ref[pl.ds(cmod(+1) * CHUNK, CHUNK), R:] = redR + shR_own
        out_ref[pl.ds(cmod(-1) * CHUNK, CHUNK), :HF] = redL + shL_own

        ar0.wait_recv()
        ar1 = hop(agR.at[0], agR.at[1], sems.at[2, 1], sems.at[3, 1], right)
        ar1.start()
        al0.wait_recv()
        al1 = hop(agL.at[0], agL.at[1], sems.at[6, 1], sems.at[7, 1], left)
        al1.start()
        sl0 = pl.ds(my_pos * CHUNK, CHUNK)
        out_ref[sl0, R:] = agR[0][:, :].astype(F32) + shared_half(my_pos, R)
        out_ref[sl0, :HF] = agL[0][:, :].astype(F32) + shared_half(my_pos, L)

        ar1.wait_recv()
        ar2 = hop(agR.at[1], agR.at[2], sems.at[2, 2], sems.at[3, 2], right)
        ar2.start()
        al1.wait_recv()
        al2 = hop(agL.at[1], agL.at[2], sems.at[6, 2], sems.at[7, 2], left)
        al2.start()
        slm1 = pl.ds(cmod(-1) * CHUNK, CHUNK)
        slp1 = pl.ds(cmod(+1) * CHUNK, CHUNK)
        out_ref[slm1, R:] = agR[1][:, :].astype(F32) + shared_half(cmod(-1), R)
        out_ref[slp1, :HF] = agL[1][:, :].astype(F32) + shared_half(cmod(+1), L)

        ar2.wait_recv()
        al2.wait_recv()
        sl2 = pl.ds(cmod(+2) * CHUNK, CHUNK)
        out_ref[sl2, R:] = agR[2][:, :].astype(F32) + shared_half(cmod(+2), R)
        out_ref[sl2, :HF] = agL[2][:, :].astype(F32) + shared_half(cmod(+2), L)

        for d in (r0, r1, r2, l0, l1, l2, ar0, ar1, ar2, al0, al1, al2):
            d.wait_send()

    return pl.pallas_call(
        body,
        out_shape=jax.ShapeDtypeStruct((N_TOK, D_FF), F32),
        in_specs=[pl.BlockSpec(memory_space=pltpu.VMEM)] * 5,
        out_specs=pl.BlockSpec(memory_space=pltpu.VMEM),
        scratch_shapes=[
            pltpu.VMEM((N_TOK, 1), F32),
            pltpu.VMEM((N_TOK, D_MODEL), BF),
            pltpu.VMEM((D_MODEL, D_FF), BF),
            pltpu.VMEM((N_DEV - 1, CHUNK, HF), BF),
            pltpu.VMEM((N_DEV - 1, CHUNK, HF), BF),
            pltpu.VMEM((CHUNK, HF), BF),
            pltpu.VMEM((N_DEV - 1, CHUNK, HF), BF),
            pltpu.VMEM((N_DEV - 1, CHUNK, HF), BF),
            pltpu.VMEM((N_DEV - 1, CHUNK, HF), BF),
            pltpu.VMEM((CHUNK, HF), BF),
            pltpu.VMEM((N_DEV - 1, CHUNK, HF), BF),
            pltpu.SemaphoreType.DMA((8, N_DEV - 1)),
        ],
        compiler_params=pltpu.CompilerParams(
            collective_id=0,
            vmem_limit_bytes=100 * 1024 * 1024,
        ),
    )(x, router_W, route_idx, expert_W, shared_W)
